# baseline (speedup 1.0000x reference)
import functools

import jax
import jax.numpy as jnp
from jax import lax
from jax.experimental import pallas as pl
from jax.experimental.pallas import tpu as pltpu
from jax.experimental.pallas import tpu_sc as plsc

D = 32
NW = 32
CHUNK_ROWS = 32
NBUF = 3
LANES = 16


def _make_lookup(B0, B1):
    rows_per_w = B0 // NW
    n_chunks = rows_per_w // CHUNK_ROWS
    chunk = CHUNK_ROWS * B1
    assert n_chunks * CHUNK_ROWS == rows_per_w
    mesh = plsc.VectorSubcoreMesh(core_axis_name="c", subcore_axis_name="s")

    @functools.partial(
        pl.kernel,
        mesh=mesh,
        out_type=jax.ShapeDtypeStruct((B0, B1, D), jnp.float32),
        scratch_types=[pltpu.VMEM((CHUNK_ROWS, B1), jnp.int32)] * NBUF
        + [pltpu.VMEM((chunk,), jnp.int32)] * NBUF
        + [pltpu.VMEM((chunk, D), jnp.float32)] * NBUF
        + [pltpu.SemaphoreType.DMA] * (2 * NBUF),
        compiler_params=pltpu.CompilerParams(
            use_tc_tiling_on_sc=False, needs_layout_passes=False),
    )
    def k(idx_hbm, table_hbm, out_hbm, *scratch):
        idx2_v = scratch[:NBUF]
        idxf_v = scratch[NBUF:2 * NBUF]
        rows_v = scratch[2 * NBUF:3 * NBUF]
        gsem = scratch[3 * NBUF:4 * NBUF]
        osem = scratch[4 * NBUF:]
        wid = lax.axis_index("s") * 2 + lax.axis_index("c")
        rbase = wid * rows_per_w
        lanes = lax.iota(jnp.int32, LANES)

        gathers = [None] * n_chunks
        stores = [None] * n_chunks

        def issue(i):
            b = i % NBUF
            if i >= NBUF:
                stores[i - NBUF].wait()
            r0 = rbase + i * CHUNK_ROWS
            pltpu.sync_copy(idx_hbm.at[pl.ds(r0, CHUNK_ROWS)], idx2_v[b])
            flat = idxf_v[b]
            idx2 = idx2_v[b]

            def fl(r, carry):
                base = r * B1 + lanes
                va = idx2[r, pl.ds(0, LANES)]
                vb = idx2[r, pl.ds(B1 - LANES, LANES)]
                plsc.store_scatter(flat, [base], va)
                plsc.store_scatter(flat, [base + (B1 - LANES)], vb)
                return carry

            lax.fori_loop(0, CHUNK_ROWS, fl, 0)
            gathers[i] = pltpu.async_copy(table_hbm.at[flat], rows_v[b],
                                          gsem[b])

        def drain(i):
            b = i % NBUF
            gathers[i].wait()
            r0 = rbase + i * CHUNK_ROWS

            def st(r, carry):
                pltpu.async_copy(rows_v[b].at[pl.ds(r * B1, B1)],
                                 out_hbm.at[r0 + r], osem[b])
                return carry

            lax.fori_loop(0, CHUNK_ROWS, st, 0)
            stores[i] = pltpu.make_async_copy(
                table_hbm.at[pl.ds(0, CHUNK_ROWS * B1)], rows_v[b], osem[b])

        for i in range(min(NBUF - 1, n_chunks)):
            issue(i)
        for i in range(n_chunks):
            if i + NBUF - 1 < n_chunks:
                issue(i + NBUF - 1)
            drain(i)
        for i in range(max(0, n_chunks - NBUF), n_chunks):
            stores[i].wait()

    return k


def kernel(token_ids, vocab):
    B0, B1 = token_ids.shape
    return _make_lookup(B0, B1)(token_ids.astype(jnp.int32), vocab)


# speedup vs baseline: 1.0001x; 1.0001x over previous
import functools

import jax
import jax.numpy as jnp
from jax import lax
from jax.experimental import pallas as pl
from jax.experimental.pallas import tpu as pltpu
from jax.experimental.pallas import tpu_sc as plsc

D = 32
NW = 32
CHUNK_ROWS = 32
NBUF = 3
LANES = 16


def _make_lookup(B0, B1):
    rows_per_w = B0 // NW
    n_chunks = rows_per_w // CHUNK_ROWS
    chunk = CHUNK_ROWS * B1
    assert n_chunks * CHUNK_ROWS == rows_per_w
    mesh = plsc.VectorSubcoreMesh(core_axis_name="c", subcore_axis_name="s")

    B1P = 32

    @functools.partial(
        pl.kernel,
        mesh=mesh,
        out_type=jax.ShapeDtypeStruct((B0, B1, D), jnp.float32),
        scratch_types=[pltpu.VMEM((CHUNK_ROWS, B1P), jnp.int32)] * NBUF
        + [pltpu.VMEM((chunk,), jnp.int32)] * NBUF
        + [pltpu.VMEM((chunk, D), jnp.float32)] * NBUF
        + [pltpu.SemaphoreType.DMA] * (2 * NBUF),
        compiler_params=pltpu.CompilerParams(
            use_tc_tiling_on_sc=False, needs_layout_passes=False),
    )
    def k(idx_hbm, table_hbm, out_hbm, *scratch):
        idx2_v = scratch[:NBUF]
        idxf_v = scratch[NBUF:2 * NBUF]
        rows_v = scratch[2 * NBUF:3 * NBUF]
        gsem = scratch[3 * NBUF:4 * NBUF]
        osem = scratch[4 * NBUF:]
        wid = lax.axis_index("s") * 2 + lax.axis_index("c")
        rbase = wid * rows_per_w
        lanes = lax.iota(jnp.int32, LANES)

        gathers = [None] * n_chunks
        stores = [None] * n_chunks

        def issue(i):
            b = i % NBUF
            if i >= NBUF:
                stores[i - NBUF].wait()
            r0 = rbase + i * CHUNK_ROWS
            pltpu.sync_copy(idx_hbm.at[pl.ds(r0, CHUNK_ROWS)], idx2_v[b])
            flat = idxf_v[b]
            idx2 = idx2_v[b]

            def fl(r, carry):
                base = r * B1 + lanes
                va = idx2[r, pl.ds(0, LANES)]
                vb = idx2[r, pl.ds(B1 - LANES, LANES)]
                plsc.store_scatter(flat, [base], va)
                plsc.store_scatter(flat, [base + (B1 - LANES)], vb)
                return carry

            lax.fori_loop(0, CHUNK_ROWS, fl, 0)
            gathers[i] = pltpu.async_copy(table_hbm.at[flat], rows_v[b],
                                          gsem[b])

        def drain(i):
            b = i % NBUF
            gathers[i].wait()
            r0 = rbase + i * CHUNK_ROWS

            def st(r, carry):
                pltpu.async_copy(rows_v[b].at[pl.ds(r * B1, B1)],
                                 out_hbm.at[r0 + r], osem[b])
                return carry

            lax.fori_loop(0, CHUNK_ROWS, st, 0)
            stores[i] = pltpu.make_async_copy(
                table_hbm.at[pl.ds(0, CHUNK_ROWS * B1)], rows_v[b], osem[b])

        for i in range(min(NBUF - 1, n_chunks)):
            issue(i)
        for i in range(n_chunks):
            if i + NBUF - 1 < n_chunks:
                issue(i + NBUF - 1)
            drain(i)
        for i in range(max(0, n_chunks - NBUF), n_chunks):
            stores[i].wait()

    return k


def kernel(token_ids, vocab):
    B0, B1 = token_ids.shape
    tp = jnp.pad(token_ids.astype(jnp.int32), ((0, 0), (0, 32 - B1)))
    return _make_lookup(B0, B1)(tp, vocab)
